# baseline (speedup 1.0000x reference)
import functools

import jax
import jax.numpy as jnp
from jax import lax
from jax.experimental import pallas as pl
from jax.experimental.pallas import tpu as pltpu
from jax.experimental.pallas import tpu_sc as plsc

B = 16384
D = 128
NC = 2
NS = 16
NW = NC * NS
CH = 128
CHUNK_SIZES = (12288, 4096)
CHUNK_STARTS = (0, 12288)
CNK = len(CHUNK_SIZES)


def _sc_gather_body(start, bpw, nch, uid_h, iid_h, ut_h, it_h, ug_h, ig_h,
                    uidx, iidx, buf0, buf1, buf2, buf3,
                    gsem0, gsem1, gsem2, gsem3, osem0, osem1, osem2, osem3):
    c = lax.axis_index("c")
    s = lax.axis_index("s")
    wid = s * NC + c
    base = wid * bpw
    np_ = 2 * nch
    pltpu.sync_copy(uid_h.at[pl.ds(start + base, bpw)], uidx)
    pltpu.sync_copy(iid_h.at[pl.ds(start + base, bpw)], iidx)

    bufs = (buf0, buf1, buf2, buf3)
    gsems = (gsem0, gsem1, gsem2, gsem3)
    osems = (osem0, osem1, osem2, osem3)
    nb = len(bufs)

    def src(p):
        if p < nch:
            return ut_h.at[uidx.at[pl.ds(p * CH, CH)]]
        return it_h.at[iidx.at[pl.ds((p - nch) * CH, CH)]]

    def dst(p):
        out = ug_h if p < nch else ig_h
        return out.at[pl.ds(base + (p % nch) * CH, CH)]

    ahead = 2 if np_ > 2 else 1
    g = [None] * nb
    o = [None] * nb
    for q in range(ahead):
        g[q] = pltpu.async_copy(src(q), bufs[q], gsems[q])
    for p in range(np_):
        cur = p % nb
        q = p + ahead
        if q < np_:
            slot = q % nb
            if o[slot] is not None:
                o[slot].wait()
                o[slot] = None
            g[slot] = pltpu.async_copy(src(q), bufs[slot], gsems[slot])
        g[cur].wait()
        o[cur] = pltpu.async_copy(bufs[cur], dst(p), osems[cur])
    for od in o:
        if od is not None:
            od.wait()


def _sc_gather(ci, uids, iids, user_table, item_table):
    cb = CHUNK_SIZES[ci]
    bpw = cb // NW
    nch = bpw // CH
    mesh = plsc.VectorSubcoreMesh(core_axis_name="c", subcore_axis_name="s",
                                  num_cores=NC, num_subcores=NS)
    run = pl.kernel(
        functools.partial(_sc_gather_body, CHUNK_STARTS[ci], bpw, nch),
        out_type=(
            jax.ShapeDtypeStruct((cb, D), jnp.float32),
            jax.ShapeDtypeStruct((cb, D), jnp.float32),
        ),
        mesh=mesh,
        compiler_params=pltpu.CompilerParams(use_tc_tiling_on_sc=True),
        scratch_types=(
            [pltpu.VMEM((bpw,), jnp.int32)] * 2
            + [pltpu.VMEM((CH, D), jnp.float32)] * 4
            + [pltpu.SemaphoreType.DMA] * 8
        ),
        name="sc_embedding_gather",
    )
    return run(uids, iids, user_table, item_table)


BLK = 4096


def _mlp_body(u_ref, i_ref, w1u_ref, w1i_ref, b1_ref, w2_ref, b2_ref,
              wp_ref, bp_ref, *rest):
    o_ref = rest[-1]
    hp = jax.lax.Precision.HIGHEST
    h = jnp.dot(u_ref[...], w1u_ref[...], precision=hp,
                preferred_element_type=jnp.float32)
    h = h + jnp.dot(i_ref[...], w1i_ref[...], precision=hp,
                    preferred_element_type=jnp.float32)
    h = jnp.maximum(h + b1_ref[...], 0.0)
    h = jnp.dot(h, w2_ref[...], precision=hp,
                preferred_element_type=jnp.float32)
    h = jnp.maximum(h + b2_ref[...], 0.0)
    o = jnp.sum(h * wp_ref[...], axis=1, keepdims=True)
    b_row = lax.broadcasted_iota(jnp.int32, (BLK, 128), 0)
    l_col = lax.broadcasted_iota(jnp.int32, (BLK, 128), 1)
    y = jnp.where(l_col == b_row % 128, o, 0.0)
    o_ref[...] = (jnp.sum(y.reshape(BLK // 128, 128, 128), axis=1)
                  + bp_ref[0, 0])


def _mlp(ci, ug, ig, W1u, W1i, b1, W2, b2, wp_row, bp, acc):
    grid = CHUNK_SIZES[ci] // BLK
    base = CHUNK_STARTS[ci] // BLK
    full = lambda shape: pl.BlockSpec(shape, lambda ib: (0, 0))
    in_specs = [
        pl.BlockSpec((BLK, D), lambda ib: (ib, 0)),
        pl.BlockSpec((BLK, D), lambda ib: (ib, 0)),
        full((D, 64)),
        full((D, 64)),
        full((1, 64)),
        full((64, 16)),
        full((1, 16)),
        full((1, 16)),
        full((1, 1)),
    ]
    args = [ug, ig, W1u, W1i, b1, W2, b2, wp_row, bp]
    aliases = {}
    if acc is not None:
        in_specs.append(pl.BlockSpec(memory_space=pl.ANY))
        args.append(acc)
        aliases = {9: 0}
    return pl.pallas_call(
        _mlp_body,
        grid=(grid,),
        in_specs=in_specs,
        out_specs=pl.BlockSpec((BLK // 128, 128),
                               lambda ib: (base + ib, 0)),
        out_shape=jax.ShapeDtypeStruct((B // 128, 128), jnp.float32),
        input_output_aliases=aliases,
        compiler_params=pltpu.CompilerParams(
            dimension_semantics=("parallel",)),
    )(*args)


@jax.jit
def kernel(U_ids, I_ids, user_table, item_table, W1, b1, W2, b2, Wp, bp):
    u32 = U_ids.astype(jnp.int32)
    i32 = I_ids.astype(jnp.int32)
    W1u = W1[:D]
    W1i = W1[D:]
    b1r = b1.reshape(1, 64)
    b2r = b2.reshape(1, 16)
    wpr = Wp.reshape(1, 16)
    bpr = bp.reshape(1, 1)
    acc = None
    for c in range(CNK):
        ug, ig = _sc_gather(c, u32, i32, user_table, item_table)
        acc = _mlp(c, ug, ig, W1u, W1i, b1r, W2, b2r, wpr, bpr, acc)
    return acc.reshape(B, 1)


# speedup vs baseline: 1.5750x; 1.5750x over previous
import functools

import jax
import jax.numpy as jnp
from jax import lax
from jax.experimental import pallas as pl
from jax.experimental.pallas import tpu as pltpu
from jax.experimental.pallas import tpu_sc as plsc

B = 16384
D = 128
NC = 2
NS = 16
NW = NC * NS
CH = 128
CHUNK_SIZES = (12288, 4096)
CHUNK_STARTS = (0, 12288)
CNK = len(CHUNK_SIZES)


def _sc_gather_body(start, bpw, nch, uid_h, iid_h, ut_h, it_h, ug_h, ig_h,
                    uidx, iidx, buf0, buf1, buf2, buf3,
                    gsem0, gsem1, gsem2, gsem3, osem0, osem1, osem2, osem3):
    c = lax.axis_index("c")
    s = lax.axis_index("s")
    wid = s * NC + c
    base = wid * bpw
    np_ = 2 * nch
    pltpu.sync_copy(uid_h.at[pl.ds(start + base, bpw)], uidx)
    pltpu.sync_copy(iid_h.at[pl.ds(start + base, bpw)], iidx)

    bufs = (buf0, buf1, buf2, buf3)
    gsems = (gsem0, gsem1, gsem2, gsem3)
    osems = (osem0, osem1, osem2, osem3)
    nb = len(bufs)

    def src(p):
        if p < nch:
            return ut_h.at[uidx.at[pl.ds(p * CH, CH)]]
        return it_h.at[iidx.at[pl.ds((p - nch) * CH, CH)]]

    def dst(p):
        out = ug_h if p < nch else ig_h
        return out.at[pl.ds(base + (p % nch) * CH, CH)]

    ahead = 2 if np_ > 2 else 1
    g = [None] * nb
    o = [None] * nb
    for q in range(ahead):
        g[q] = pltpu.async_copy(src(q), bufs[q], gsems[q])
    for p in range(np_):
        cur = p % nb
        q = p + ahead
        if q < np_:
            slot = q % nb
            if o[slot] is not None:
                o[slot].wait()
                o[slot] = None
            g[slot] = pltpu.async_copy(src(q), bufs[slot], gsems[slot])
        g[cur].wait()
        o[cur] = pltpu.async_copy(bufs[cur], dst(p), osems[cur])
    for od in o:
        if od is not None:
            od.wait()


def _sc_gather(ci, uids, iids, user_table, item_table):
    cb = CHUNK_SIZES[ci]
    bpw = cb // NW
    nch = bpw // CH
    mesh = plsc.VectorSubcoreMesh(core_axis_name="c", subcore_axis_name="s",
                                  num_cores=NC, num_subcores=NS)
    run = pl.kernel(
        functools.partial(_sc_gather_body, CHUNK_STARTS[ci], bpw, nch),
        out_type=(
            jax.ShapeDtypeStruct((cb, D), jnp.float32),
            jax.ShapeDtypeStruct((cb, D), jnp.float32),
        ),
        mesh=mesh,
        compiler_params=pltpu.CompilerParams(use_tc_tiling_on_sc=True),
        scratch_types=(
            [pltpu.VMEM((bpw,), jnp.int32)] * 2
            + [pltpu.VMEM((CH, D), jnp.float32)] * 4
            + [pltpu.SemaphoreType.DMA] * 8
        ),
        name="sc_embedding_gather",
    )
    return run(uids, iids, user_table, item_table)


BLK = 4096


def _mlp_body(u_ref, i_ref, w1u_ref, w1i_ref, b1_ref, w2_ref, b2_ref,
              wp_ref, bp_ref, *rest):
    o_ref = rest[-1]
    h = jnp.dot(u_ref[...], w1u_ref[...], preferred_element_type=jnp.float32)
    h = h + jnp.dot(i_ref[...], w1i_ref[...], preferred_element_type=jnp.float32)
    h = jnp.maximum(h + b1_ref[...], 0.0)
    h = jnp.dot(h, w2_ref[...], preferred_element_type=jnp.float32)
    h = jnp.maximum(h + b2_ref[...], 0.0)
    o = jnp.sum(h * wp_ref[...], axis=1, keepdims=True)
    b_row = lax.broadcasted_iota(jnp.int32, (BLK, 128), 0)
    l_col = lax.broadcasted_iota(jnp.int32, (BLK, 128), 1)
    y = jnp.where(l_col == b_row % 128, o, 0.0)
    o_ref[...] = (jnp.sum(y.reshape(BLK // 128, 128, 128), axis=1)
                  + bp_ref[0, 0])


def _mlp(ci, ug, ig, W1u, W1i, b1, W2, b2, wp_row, bp, acc):
    grid = CHUNK_SIZES[ci] // BLK
    base = CHUNK_STARTS[ci] // BLK
    full = lambda shape: pl.BlockSpec(shape, lambda ib: (0, 0))
    in_specs = [
        pl.BlockSpec((BLK, D), lambda ib: (ib, 0)),
        pl.BlockSpec((BLK, D), lambda ib: (ib, 0)),
        full((D, 64)),
        full((D, 64)),
        full((1, 64)),
        full((64, 16)),
        full((1, 16)),
        full((1, 16)),
        full((1, 1)),
    ]
    args = [ug, ig, W1u, W1i, b1, W2, b2, wp_row, bp]
    aliases = {}
    if acc is not None:
        in_specs.append(pl.BlockSpec(memory_space=pl.ANY))
        args.append(acc)
        aliases = {9: 0}
    return pl.pallas_call(
        _mlp_body,
        grid=(grid,),
        in_specs=in_specs,
        out_specs=pl.BlockSpec((BLK // 128, 128),
                               lambda ib: (base + ib, 0)),
        out_shape=jax.ShapeDtypeStruct((B // 128, 128), jnp.float32),
        input_output_aliases=aliases,
        compiler_params=pltpu.CompilerParams(
            dimension_semantics=("parallel",)),
    )(*args)


@jax.jit
def kernel(U_ids, I_ids, user_table, item_table, W1, b1, W2, b2, Wp, bp):
    u32 = U_ids.astype(jnp.int32)
    i32 = I_ids.astype(jnp.int32)
    W1u = W1[:D]
    W1i = W1[D:]
    b1r = b1.reshape(1, 64)
    b2r = b2.reshape(1, 16)
    wpr = Wp.reshape(1, 16)
    bpr = bp.reshape(1, 1)
    acc = None
    for c in range(CNK):
        ug, ig = _sc_gather(c, u32, i32, user_table, item_table)
        acc = _mlp(c, ug, ig, W1u, W1i, b1r, W2, b2r, wpr, bpr, acc)
    return acc.reshape(B, 1)
